# initial kernel scaffold (unmeasured)
import jax
import jax.numpy as jnp
from jax import lax
from jax.experimental import pallas as pl
from jax.experimental.pallas import tpu as pltpu


def kernel(
    x,
):
    def body(*refs):
        pass

    out_shape = jax.ShapeDtypeStruct(..., jnp.float32)
    return pl.pallas_call(body, out_shape=out_shape)(...)


# baseline (device time: 71123 ns/iter reference)
import jax
import jax.numpy as jnp
from jax import lax
from jax.experimental import pallas as pl
from jax.experimental.pallas import tpu as pltpu

N_DEV = 16
M = 512
N = 512
CHUNK = M // N_DEV


def kernel(x):
    def body(
        x_ref,
        out_ref,
        send_buf,
        rs_recv_buf,
        ag_recv_buf,
        rs_send_sems,
        rs_recv_sems,
        ag_send_sems,
        ag_recv_sems,
    ):
        my = lax.axis_index("i")
        left = (my + N_DEV - 1) % N_DEV
        right = (my + 1) % N_DEV

        barrier = pltpu.get_barrier_semaphore()
        for nbr in (left, right):
            pl.semaphore_signal(
                barrier, inc=1,
                device_id=(nbr,), device_id_type=pl.DeviceIdType.MESH,
            )
        pl.semaphore_wait(barrier, 2)

        for s in range(N_DEV - 1):
            row = ((my - s) % N_DEV) * CHUNK
            local = x_ref[pl.ds(row, CHUNK), :].astype(jnp.bfloat16)
            if s == 0:
                send_buf[s, :, :] = local
            else:
                send_buf[s, :, :] = rs_recv_buf[s - 1, :, :] + local
            rdma = pltpu.make_async_remote_copy(
                src_ref=send_buf.at[s],
                dst_ref=rs_recv_buf.at[s],
                send_sem=rs_send_sems.at[s],
                recv_sem=rs_recv_sems.at[s],
                device_id=(right,),
                device_id_type=pl.DeviceIdType.MESH,
            )
            rdma.start()
            rdma.wait()

        own = (my + 1) % N_DEV
        own_row = own * CHUNK
        reduced = (
            rs_recv_buf[N_DEV - 2, :, :]
            + x_ref[pl.ds(own_row, CHUNK), :].astype(jnp.bfloat16)
        )
        send_buf[N_DEV - 1, :, :] = reduced
        out_ref[pl.ds(own_row, CHUNK), :] = reduced.astype(jnp.float32)

        for t in range(N_DEV - 1):
            src = send_buf.at[N_DEV - 1] if t == 0 else ag_recv_buf.at[t - 1]
            rdma = pltpu.make_async_remote_copy(
                src_ref=src,
                dst_ref=ag_recv_buf.at[t],
                send_sem=ag_send_sems.at[t],
                recv_sem=ag_recv_sems.at[t],
                device_id=(right,),
                device_id_type=pl.DeviceIdType.MESH,
            )
            rdma.start()
            rdma.wait()
            dst_row = ((my - t) % N_DEV) * CHUNK
            out_ref[pl.ds(dst_row, CHUNK), :] = (
                ag_recv_buf[t, :, :].astype(jnp.float32)
            )

    return pl.pallas_call(
        body,
        out_shape=jax.ShapeDtypeStruct((M, N), jnp.float32),
        in_specs=[pl.BlockSpec(memory_space=pltpu.VMEM)],
        out_specs=pl.BlockSpec(memory_space=pltpu.VMEM),
        scratch_shapes=[
            pltpu.VMEM((N_DEV, CHUNK, N), jnp.bfloat16),
            pltpu.VMEM((N_DEV - 1, CHUNK, N), jnp.bfloat16),
            pltpu.VMEM((N_DEV - 1, CHUNK, N), jnp.bfloat16),
            pltpu.SemaphoreType.DMA((N_DEV - 1,)),
            pltpu.SemaphoreType.DMA((N_DEV - 1,)),
            pltpu.SemaphoreType.DMA((N_DEV - 1,)),
            pltpu.SemaphoreType.DMA((N_DEV - 1,)),
        ],
        compiler_params=pltpu.CompilerParams(collective_id=0),
    )(x)


# device time: 21621 ns/iter; 3.2895x vs baseline; 3.2895x over previous
import jax
import jax.numpy as jnp
from jax import lax
from jax.experimental import pallas as pl
from jax.experimental.pallas import tpu as pltpu

N_DEV = 16
M = 512
N = 512
CHUNK = M // N_DEV


def kernel(x):
    def body(
        x_ref,
        out_ref,
        xbf,
        ag_src,
        rs_buf,
        ag_buf,
        rs_send_sems,
        rs_recv_sems,
        ag_send_sems,
        ag_recv_sems,
    ):
        my = lax.axis_index("i")

        barrier = pltpu.get_barrier_semaphore()
        for off in range(1, N_DEV):
            pl.semaphore_signal(
                barrier, inc=1,
                device_id=((my + off) % N_DEV,),
                device_id_type=pl.DeviceIdType.MESH,
            )
        pl.semaphore_wait(barrier, N_DEV - 1)

        xbf[:, :] = x_ref[:, :].astype(jnp.bfloat16)

        rs = []
        for off in range(1, N_DEV):
            s = N_DEV - off
            tgt = (my + off) % N_DEV
            d = pltpu.make_async_remote_copy(
                src_ref=xbf.at[pl.ds(tgt * CHUNK, CHUNK), :],
                dst_ref=rs_buf.at[s],
                send_sem=rs_send_sems.at[s],
                recv_sem=rs_recv_sems.at[s],
                device_id=(tgt,),
                device_id_type=pl.DeviceIdType.MESH,
            )
            d.start()
            rs.append((s, d))

        reduced = x_ref[pl.ds(my * CHUNK, CHUNK), :].astype(jnp.bfloat16)
        for s, d in rs:
            d.wait_recv()
            reduced = reduced + rs_buf[s, :, :]

        ag_src[:, :] = reduced
        out_ref[pl.ds(my * CHUNK, CHUNK), :] = reduced.astype(jnp.float32)

        ag = []
        for off in range(1, N_DEV):
            s = N_DEV - off
            tgt = (my + off) % N_DEV
            d = pltpu.make_async_remote_copy(
                src_ref=ag_src,
                dst_ref=ag_buf.at[s],
                send_sem=ag_send_sems.at[s],
                recv_sem=ag_recv_sems.at[s],
                device_id=(tgt,),
                device_id_type=pl.DeviceIdType.MESH,
            )
            d.start()
            ag.append((s, d))

        for s, d in ag:
            d.wait_recv()
            row = ((my + s) % N_DEV) * CHUNK
            out_ref[pl.ds(row, CHUNK), :] = ag_buf[s, :, :].astype(jnp.float32)

        for _, d in rs:
            d.wait_send()
        for _, d in ag:
            d.wait_send()

    return pl.pallas_call(
        body,
        out_shape=jax.ShapeDtypeStruct((M, N), jnp.float32),
        in_specs=[pl.BlockSpec(memory_space=pltpu.VMEM)],
        out_specs=pl.BlockSpec(memory_space=pltpu.VMEM),
        scratch_shapes=[
            pltpu.VMEM((M, N), jnp.bfloat16),
            pltpu.VMEM((CHUNK, N), jnp.bfloat16),
            pltpu.VMEM((N_DEV, CHUNK, N), jnp.bfloat16),
            pltpu.VMEM((N_DEV, CHUNK, N), jnp.bfloat16),
            pltpu.SemaphoreType.DMA((N_DEV,)),
            pltpu.SemaphoreType.DMA((N_DEV,)),
            pltpu.SemaphoreType.DMA((N_DEV,)),
            pltpu.SemaphoreType.DMA((N_DEV,)),
        ],
        compiler_params=pltpu.CompilerParams(collective_id=0),
    )(x)


# device time: 21526 ns/iter; 3.3041x vs baseline; 1.0044x over previous
import jax
import jax.numpy as jnp
from jax import lax
from jax.experimental import pallas as pl
from jax.experimental.pallas import tpu as pltpu

N_DEV = 16
M = 512
N = 512
CHUNK = M // N_DEV


def kernel(x):
    def body(
        x_ref,
        out_ref,
        xbf,
        ag_src,
        rs_buf,
        ag_buf,
        rs_send_sems,
        rs_recv_sems,
        ag_send_sems,
        ag_recv_sems,
    ):
        my = lax.axis_index("i")

        barrier = pltpu.get_barrier_semaphore()
        for off in range(1, N_DEV):
            pl.semaphore_signal(
                barrier, inc=1,
                device_id=((my + off) % N_DEV,),
                device_id_type=pl.DeviceIdType.MESH,
            )
        xbf[:, :] = x_ref[:, :].astype(jnp.bfloat16)
        pl.semaphore_wait(barrier, N_DEV - 1)

        rs = []
        for off in range(1, N_DEV):
            s = N_DEV - off
            tgt = (my + off) % N_DEV
            d = pltpu.make_async_remote_copy(
                src_ref=xbf.at[pl.ds(tgt * CHUNK, CHUNK), :],
                dst_ref=rs_buf.at[s],
                send_sem=rs_send_sems.at[s],
                recv_sem=rs_recv_sems.at[s],
                device_id=(tgt,),
                device_id_type=pl.DeviceIdType.MESH,
            )
            d.start()
            rs.append((s, d))

        reduced = x_ref[pl.ds(my * CHUNK, CHUNK), :].astype(jnp.bfloat16)
        for s, d in rs:
            d.wait_recv()
            reduced = reduced + rs_buf[s, :, :]

        ag_src[:, :] = reduced

        ag = []
        for off in range(1, N_DEV):
            s = N_DEV - off
            tgt = (my + off) % N_DEV
            d = pltpu.make_async_remote_copy(
                src_ref=ag_src,
                dst_ref=ag_buf.at[s],
                send_sem=ag_send_sems.at[s],
                recv_sem=ag_recv_sems.at[s],
                device_id=(tgt,),
                device_id_type=pl.DeviceIdType.MESH,
            )
            d.start()
            ag.append((s, d))

        out_ref[pl.ds(my * CHUNK, CHUNK), :] = reduced.astype(jnp.float32)

        for s, d in ag:
            d.wait_recv()
            row = ((my + s) % N_DEV) * CHUNK
            out_ref[pl.ds(row, CHUNK), :] = ag_buf[s, :, :].astype(jnp.float32)

        for _, d in rs:
            d.wait_send()
        for _, d in ag:
            d.wait_send()

    return pl.pallas_call(
        body,
        out_shape=jax.ShapeDtypeStruct((M, N), jnp.float32),
        in_specs=[pl.BlockSpec(memory_space=pltpu.VMEM)],
        out_specs=pl.BlockSpec(memory_space=pltpu.VMEM),
        scratch_shapes=[
            pltpu.VMEM((M, N), jnp.bfloat16),
            pltpu.VMEM((CHUNK, N), jnp.bfloat16),
            pltpu.VMEM((N_DEV, CHUNK, N), jnp.bfloat16),
            pltpu.VMEM((N_DEV, CHUNK, N), jnp.bfloat16),
            pltpu.SemaphoreType.DMA((N_DEV,)),
            pltpu.SemaphoreType.DMA((N_DEV,)),
            pltpu.SemaphoreType.DMA((N_DEV,)),
            pltpu.SemaphoreType.DMA((N_DEV,)),
        ],
        compiler_params=pltpu.CompilerParams(collective_id=0),
    )(x)
